# baseline (speedup 1.0000x reference)
import dataclasses
import functools

import jax
import jax.numpy as jnp
from jax import lax
from jax.experimental import pallas as pl
from jax.experimental.pallas import tpu as pltpu
from jax.experimental.pallas import tpu_sc as plsc

_NC = 2
_NS = 16
_NW = _NC * _NS
_L = 16

_BATCH = 16384
_D = 16
_B_PER_W = _BATCH // _NW
_CHUNK = 128
_ROWS_PER_BIG = 128 // _D


def _compiler_params():
    cp = pltpu.CompilerParams()
    if "needs_layout_passes" in pltpu.CompilerParams.__dataclass_fields__:
        cp = dataclasses.replace(cp, needs_layout_passes=False)
    return cp


def kernel(user_ids, item_ids, user_table, item_table):
    n_users, d = user_table.shape
    ut_big = user_table.reshape(n_users * d // 128, 128)
    it_big = item_table.reshape(item_table.shape[0] * d // 128, 128)

    mesh = plsc.VectorSubcoreMesh(core_axis_name="c", subcore_axis_name="s")

    @functools.partial(
        pl.kernel,
        mesh=mesh,
        out_type=jax.ShapeDtypeStruct((_BATCH,), jnp.float32),
        scratch_types=[
            pltpu.VMEM((_B_PER_W,), jnp.int32),
            pltpu.VMEM((_B_PER_W,), jnp.int32),
            pltpu.VMEM((_B_PER_W,), jnp.int32),
            pltpu.VMEM((_B_PER_W,), jnp.int32),
            pltpu.VMEM((_CHUNK, 128), jnp.float32),
            pltpu.VMEM((_CHUNK, 128), jnp.float32),
            pltpu.VMEM((_B_PER_W,), jnp.float32),
            pltpu.SemaphoreType.DMA,
            pltpu.SemaphoreType.DMA,
        ],
        compiler_params=_compiler_params(),
    )
    def sc_kernel(uid_hbm, iid_hbm, ut_hbm, it_hbm, out_hbm,
                  uidx_v, iidx_v, ubig_v, ibig_v, u_rows, i_rows, out_v,
                  sem_u, sem_i):
        wid = lax.axis_index("s") * _NC + lax.axis_index("c")
        base = wid * _B_PER_W
        pltpu.sync_copy(uid_hbm.at[pl.ds(base, _B_PER_W)], uidx_v)
        pltpu.sync_copy(iid_hbm.at[pl.ds(base, _B_PER_W)], iidx_v)

        @pl.loop(0, _B_PER_W, step=_L)
        def _(k):
            ubig_v[pl.ds(k, _L)] = uidx_v[pl.ds(k, _L)] >> 3
            ibig_v[pl.ds(k, _L)] = iidx_v[pl.ds(k, _L)] >> 3

        lane = lax.iota(jnp.int32, _L)

        for c in range(_B_PER_W // _CHUNK):
            cu = pltpu.async_copy(
                ut_hbm.at[ubig_v.at[pl.ds(c * _CHUNK, _CHUNK)]], u_rows, sem_u)
            ci = pltpu.async_copy(
                it_hbm.at[ibig_v.at[pl.ds(c * _CHUNK, _CHUNK)]], i_rows, sem_i)
            cu.wait()
            ci.wait()

            @pl.loop(0, _CHUNK, step=_L)
            def _(g):
                j = g + lane
                uid = uidx_v[pl.ds(c * _CHUNK + g, _L)]
                iid = iidx_v[pl.ds(c * _CHUNK + g, _L)]
                ucol = (uid & (_ROWS_PER_BIG - 1)) * _D
                icol = (iid & (_ROWS_PER_BIG - 1)) * _D
                acc = jnp.zeros((_L,), jnp.float32)
                for dd in range(_D):
                    ug = plsc.load_gather(u_rows, [j, ucol + dd])
                    vg = plsc.load_gather(i_rows, [j, icol + dd])
                    acc = acc + ug * vg
                out_v[pl.ds(c * _CHUNK + g, _L)] = acc

        pltpu.sync_copy(out_v, out_hbm.at[pl.ds(base, _B_PER_W)])

    return sc_kernel(user_ids, item_ids, ut_big, it_big)
